# Initial kernel scaffold; baseline (speedup 1.0000x reference)
import jax
import jax.numpy as jnp
from jax.experimental import pallas as pl


def kernel(x, edge_index, batch, W1, b1, W2, b2, Wfc, bfc):
    raise NotImplementedError("write your pallas kernel here")


# speedup vs baseline: 8.7890x; 8.7890x over previous
import functools

import jax
import jax.numpy as jnp
from jax import lax
from jax.experimental import pallas as pl
from jax.experimental.pallas import tpu as pltpu
from jax.experimental.pallas import tpu_sc as plsc

N = 10000
NP = 10240
E = 320000
EP = 327680
D = 128
B = 64
NC = 2
NS = 16
NW = NC * NS
CH = 128
CHUNKS = EP // (NW * CH)
RPT = NP // NS
BLK = 512
NBLK = NP // BLK


def _sc_mesh():
    return plsc.VectorSubcoreMesh(core_axis_name="c", subcore_axis_name="s")


@functools.partial(
    pl.kernel,
    out_type=(jax.ShapeDtypeStruct((NP,), jnp.float32),
              jax.ShapeDtypeStruct((NP,), jnp.float32)),
    mesh=_sc_mesh(),
    scratch_types=[
        pltpu.VMEM((CHUNKS, CH), jnp.int32),
        pltpu.VMEM((CH,), jnp.float32),
        pltpu.VMEM_SHARED((NP,), jnp.float32),
    ],
)
def _sc_degree(dst_hbm, zrow_hbm, ones_hbm, out0, out1, dst_v, ones_v, acc_sh):
    c = lax.axis_index("c")
    s = lax.axis_index("s")
    wid = c * NS + s
    pltpu.sync_copy(dst_hbm.at[wid], dst_v)
    pltpu.sync_copy(ones_hbm, ones_v)
    pltpu.sync_copy(zrow_hbm, acc_sh.at[pl.ds(s * RPT, RPT)])
    plsc.subcore_barrier()

    def body(j, carry):
        pltpu.sync_copy(ones_v, acc_sh.at[dst_v.at[j]], add=True)
        return carry

    lax.fori_loop(0, CHUNKS, body, 0)
    plsc.subcore_barrier()

    @pl.when(c == 0)
    def _():
        pltpu.sync_copy(acc_sh.at[pl.ds(s * RPT, RPT)],
                        out0.at[pl.ds(s * RPT, RPT)])

    @pl.when(c == 1)
    def _():
        pltpu.sync_copy(acc_sh.at[pl.ds(s * RPT, RPT)],
                        out1.at[pl.ds(s * RPT, RPT)])


@functools.partial(
    pl.kernel,
    out_type=(jax.ShapeDtypeStruct((NP, D), jnp.float32),
              jax.ShapeDtypeStruct((NP, D), jnp.float32)),
    mesh=_sc_mesh(),
    scratch_types=[
        pltpu.VMEM((CHUNKS, CH), jnp.int32),
        pltpu.VMEM((CHUNKS, CH), jnp.int32),
        pltpu.VMEM((CH, D), jnp.float32),
        pltpu.VMEM_SHARED((NP, D), jnp.float32),
        pltpu.SemaphoreType.DMA,
    ],
)
def _sc_aggregate(hs_hbm, src_hbm, dst_hbm, zblk_hbm, out0, out1,
                  src_v, dst_v, buf, acc_sh, sem):
    c = lax.axis_index("c")
    s = lax.axis_index("s")
    wid = c * NS + s
    pltpu.sync_copy(src_hbm.at[wid], src_v)
    pltpu.sync_copy(dst_hbm.at[wid], dst_v)
    pltpu.sync_copy(zblk_hbm, acc_sh.at[pl.ds(s * RPT, RPT), :])
    plsc.subcore_barrier()

    def body(j, carry):
        pltpu.async_copy(hs_hbm.at[src_v.at[j]], buf, sem).wait()
        pltpu.sync_copy(buf, acc_sh.at[dst_v.at[j]], add=True)
        return carry

    lax.fori_loop(0, CHUNKS, body, 0)
    plsc.subcore_barrier()

    @pl.when(c == 0)
    def _():
        pltpu.sync_copy(acc_sh.at[pl.ds(s * RPT, RPT), :],
                        out0.at[pl.ds(s * RPT, RPT), :])

    @pl.when(c == 1)
    def _():
        pltpu.sync_copy(acc_sh.at[pl.ds(s * RPT, RPT), :],
                        out1.at[pl.ds(s * RPT, RPT), :])


def _prep1_body(x_ref, w1_ref, d0_ref, d1_ref, h1s_ref, dinv_ref):
    deg = d0_ref[...] + d1_ref[...] + 1.0
    dinv = lax.rsqrt(deg)
    h1 = lax.dot_general(x_ref[...], w1_ref[...], (((1,), (1,)), ((), ())),
                         preferred_element_type=jnp.float32)
    h1s_ref[...] = h1 * dinv
    dinv_ref[...] = dinv


def _mid_body(a0_ref, a1_ref, hs_ref, dinv_ref, b_ref, w_ref, out_ref):
    dinv = dinv_ref[...]
    z = dinv * (a0_ref[...] + a1_ref[...] + hs_ref[...]) + b_ref[...]
    z = jnp.where(z >= 0, z, 0.01 * z)
    h2 = lax.dot_general(z, w_ref[...], (((1,), (1,)), ((), ())),
                         preferred_element_type=jnp.float32)
    out_ref[...] = h2 * dinv


def _final_body(a0_ref, a1_ref, hs_ref, dinv_ref, b_ref, batch_ref,
                wfc_ref, bfc_ref, out_ref, pooled_acc, cnt_acc):
    i = pl.program_id(0)
    dinv = dinv_ref[...]
    z = dinv * (a0_ref[...] + a1_ref[...] + hs_ref[...]) + b_ref[...]
    z = jnp.where(z >= 0, z, 0.01 * z)
    oh = (batch_ref[...] == lax.broadcasted_iota(jnp.int32, (BLK, B), 1)
          ).astype(jnp.float32)
    p = lax.dot_general(oh, z, (((0,), (0,)), ((), ())),
                        preferred_element_type=jnp.float32)
    cnt = lax.dot_general(oh, jnp.ones((BLK, 1), jnp.float32),
                          (((0,), (0,)), ((), ())),
                          preferred_element_type=jnp.float32)

    @pl.when(i == 0)
    def _():
        pooled_acc[...] = p
        cnt_acc[...] = cnt

    @pl.when(i > 0)
    def _():
        pooled_acc[...] += p
        cnt_acc[...] += cnt

    @pl.when(i == pl.num_programs(0) - 1)
    def _():
        pooled = pooled_acc[...] / jnp.maximum(cnt_acc[...], 1.0)
        out_ref[...] = lax.dot_general(
            pooled, wfc_ref[...], (((1,), (1,)), ((), ())),
            preferred_element_type=jnp.float32) + bfc_ref[...]


def _row_spec():
    return pl.BlockSpec((BLK, D), lambda i: (i, 0))


def _col_spec():
    return pl.BlockSpec((BLK, 1), lambda i: (i, 0))


def _full_spec(shape):
    return pl.BlockSpec(shape, lambda i: tuple(0 for _ in shape))


def kernel(x, edge_index, batch, W1, b1, W2, b2, Wfc, bfc):
    f32 = jnp.float32
    src = edge_index[0].astype(jnp.int32)
    dst = edge_index[1].astype(jnp.int32)
    pad_e = EP - E
    src_p = jnp.concatenate([src, jnp.zeros((pad_e,), jnp.int32)])
    dst_p = jnp.concatenate(
        [dst, (N + jnp.arange(pad_e, dtype=jnp.int32) % (NP - N))])
    src_t = src_p.reshape(NW, CHUNKS, CH)
    dst_t = dst_p.reshape(NW, CHUNKS, CH)
    x_p = jnp.pad(x, ((0, NP - N), (0, 0)))
    batch_p = jnp.pad(batch.astype(jnp.int32), (0, NP - N),
                      constant_values=B).reshape(NP, 1)
    zrow = jnp.zeros((RPT,), f32)
    zblk = jnp.zeros((RPT, D), f32)
    ones_ch = jnp.ones((CH,), f32)

    deg0, deg1 = _sc_degree(dst_t, zrow, ones_ch)

    h1s, dinv = pl.pallas_call(
        _prep1_body,
        grid=(NBLK,),
        in_specs=[_row_spec(), _full_spec((D, D)), _col_spec(), _col_spec()],
        out_specs=[_row_spec(), _col_spec()],
        out_shape=[jax.ShapeDtypeStruct((NP, D), f32),
                   jax.ShapeDtypeStruct((NP, 1), f32)],
    )(x_p, W1, deg0.reshape(NP, 1), deg1.reshape(NP, 1))

    acc10, acc11 = _sc_aggregate(h1s, src_t, dst_t, zblk)

    h2s = pl.pallas_call(
        _mid_body,
        grid=(NBLK,),
        in_specs=[_row_spec(), _row_spec(), _row_spec(), _col_spec(),
                  _full_spec((1, D)), _full_spec((D, D))],
        out_specs=_row_spec(),
        out_shape=jax.ShapeDtypeStruct((NP, D), f32),
    )(acc10, acc11, h1s, dinv, b1.reshape(1, D), W2)

    acc20, acc21 = _sc_aggregate(h2s, src_t, dst_t, zblk)

    logits = pl.pallas_call(
        _final_body,
        grid=(NBLK,),
        in_specs=[_row_spec(), _row_spec(), _row_spec(), _col_spec(),
                  _full_spec((1, D)), _col_spec(),
                  _full_spec((2, D)), _full_spec((1, 2))],
        out_specs=pl.BlockSpec((B, 2), lambda i: (0, 0)),
        out_shape=jax.ShapeDtypeStruct((B, 2), f32),
        scratch_shapes=[pltpu.VMEM((B, D), f32), pltpu.VMEM((B, 1), f32)],
    )(acc20, acc21, h2s, dinv, b2.reshape(1, D), batch_p,
      Wfc, bfc.reshape(1, 2))

    return logits
